# baseline (speedup 1.0000x reference)
import functools

import jax
import jax.numpy as jnp
from jax import lax
from jax.experimental import pallas as pl
from jax.experimental.pallas import tpu as pltpu
from jax.experimental.pallas import tpu_sc as plsc

N_NODES = 10000
N_EDGES = 320000
D = 128
NC, NS = 2, 16
NW = NC * NS
NPAD = NS * 640
CW = 40
CHUNKS_PER_W = N_EDGES // (NW * CW)
NH = 5
CHUNKS_H = CHUNKS_PER_W // NH
EPW = N_EDGES // NW
EPH = EPW // NH
RPS = NPAD // NS

_vmesh = plsc.VectorSubcoreMesh(core_axis_name="c", subcore_axis_name="s")


@functools.partial(
    pl.kernel,
    out_type=jax.ShapeDtypeStruct((NC * NPAD,), jnp.float32),
    mesh=_vmesh,
    scratch_types=[
        pltpu.MemorySpace.VMEM_SHARED((NPAD,), jnp.float32),
        pltpu.MemorySpace.VMEM((EPH,), jnp.int32),
        pltpu.MemorySpace.VMEM((CW,), jnp.float32),
    ],
)
def _deg_kernel(dst_hbm, zeros_hbm, ones_hbm, deg_hbm, acc, idx_v, ones_v):
    c = lax.axis_index("c")
    s = lax.axis_index("s")
    wid = c * NS + s
    pltpu.sync_copy(zeros_hbm, acc.at[pl.ds(s * RPS, RPS)])
    pltpu.sync_copy(ones_hbm, ones_v)
    plsc.subcore_barrier()

    for h in range(NH):
        pltpu.sync_copy(dst_hbm.at[pl.ds(wid * EPW + h * EPH, EPH)], idx_v)

        @pl.loop(0, CHUNKS_H)
        def _edge_chunk(j):
            pltpu.sync_copy(ones_v, acc.at[idx_v.at[pl.ds(j * CW, CW)]],
                            add=True)

    plsc.subcore_barrier()
    pltpu.sync_copy(acc.at[pl.ds(s * RPS, RPS)],
                    deg_hbm.at[pl.ds(c * NPAD + s * RPS, RPS)])


NB = 2


@functools.partial(
    pl.kernel,
    out_type=jax.ShapeDtypeStruct((NC, NPAD, D), jnp.float32),
    mesh=_vmesh,
    scratch_types=[
        pltpu.MemorySpace.VMEM_SHARED((NPAD, D), jnp.float32),
        pltpu.MemorySpace.VMEM((EPH,), jnp.int32),
        pltpu.MemorySpace.VMEM((EPH,), jnp.int32),
        pltpu.MemorySpace.VMEM((NB, CW, D), jnp.float32),
        pltpu.SemaphoreType.DMA((NB,)),
    ],
)
def _agg_kernel(y_hbm, src_hbm, dst_hbm, zrows_hbm, out_hbm,
                acc, src_v, dst_v, rows_v, gsems):
    c = lax.axis_index("c")
    s = lax.axis_index("s")
    wid = c * NS + s
    pltpu.sync_copy(zrows_hbm, acc.at[pl.ds(s * RPS, RPS), :])
    plsc.subcore_barrier()

    for h in range(NH):
        pltpu.sync_copy(src_hbm.at[pl.ds(wid * EPW + h * EPH, EPH)], src_v)
        pltpu.sync_copy(dst_hbm.at[pl.ds(wid * EPW + h * EPH, EPH)], dst_v)
        for b in range(NB):
            pltpu.async_copy(y_hbm.at[src_v.at[pl.ds(b * CW, CW)]],
                             rows_v.at[b], gsems.at[b])

        @pl.loop(0, CHUNKS_H, step=NB)
        def _edge_chunk(j):
            for b in range(NB):
                pltpu.make_async_copy(
                    y_hbm.at[src_v.at[pl.ds(b * CW, CW)]], rows_v.at[b],
                    gsems.at[b]).wait()
                pltpu.sync_copy(rows_v.at[b],
                                acc.at[dst_v.at[pl.ds((j + b) * CW, CW)]],
                                add=True)

                @pl.when(j + NB + b < CHUNKS_H)
                def _refill():
                    pltpu.async_copy(
                        y_hbm.at[src_v.at[pl.ds((j + NB + b) * CW, CW)]],
                        rows_v.at[b], gsems.at[b])

    plsc.subcore_barrier()
    pltpu.sync_copy(acc.at[pl.ds(s * RPS, RPS), :],
                    out_hbm.at[c, pl.ds(s * RPS, RPS), :])


def _xw_body(x_ref, w_ref, deg_ref, y_ref):
    xw = jnp.dot(x_ref[...], w_ref[...], preferred_element_type=jnp.float32)
    d = deg_ref[0, :] + deg_ref[1, :]
    inv = lax.rsqrt(jnp.maximum(d, 1.0))
    y_ref[...] = xw * inv[:, None]


def _fin_body(acc_ref, deg_ref, out_ref):
    a = acc_ref[0, :, :] + acc_ref[1, :, :]
    d = deg_ref[0, :] + deg_ref[1, :]
    inv = lax.rsqrt(jnp.maximum(d, 1.0))
    out_ref[...] = a * inv[:, None]


_BT = 1024
_BF = 1024

def kernel(X, edge_index, W):
    src = edge_index[0]
    dst = edge_index[1]
    zeros1 = jnp.zeros((RPS,), jnp.float32)
    ones1 = jnp.ones((CW,), jnp.float32)
    zrows = jnp.zeros((RPS, D), jnp.float32)

    degp = _deg_kernel(dst, zeros1, ones1).reshape(NC, NPAD)

    y = pl.pallas_call(
        _xw_body,
        grid=(pl.cdiv(N_NODES, _BT),),
        in_specs=[
            pl.BlockSpec((_BT, D), lambda i: (i, 0)),
            pl.BlockSpec((D, D), lambda i: (0, 0)),
            pl.BlockSpec((NC, _BT), lambda i: (0, i)),
        ],
        out_specs=pl.BlockSpec((_BT, D), lambda i: (i, 0)),
        out_shape=jax.ShapeDtypeStruct((N_NODES, D), jnp.float32),
    )(X, W, degp)

    accp = _agg_kernel(y, src, dst, zrows)

    out = pl.pallas_call(
        _fin_body,
        grid=(pl.cdiv(N_NODES, _BF),),
        in_specs=[
            pl.BlockSpec((NC, _BF, D), lambda i: (0, i, 0)),
            pl.BlockSpec((NC, _BF), lambda i: (0, i)),
        ],
        out_specs=pl.BlockSpec((_BF, D), lambda i: (i, 0)),
        out_shape=jax.ShapeDtypeStruct((N_NODES, D), jnp.float32),
    )(accp, degp)

    return out


# speedup vs baseline: 1.1265x; 1.1265x over previous
import functools

import jax
import jax.numpy as jnp
from jax import lax
from jax.experimental import pallas as pl
from jax.experimental.pallas import tpu as pltpu
from jax.experimental.pallas import tpu_sc as plsc

N_NODES = 10000
N_EDGES = 320000
D = 128
NC, NS = 2, 16
NW = NC * NS
NPAD = NS * 640
CW = 125
CHUNKS_PER_W = N_EDGES // (NW * CW)
NH = 4
CHUNKS_H = CHUNKS_PER_W // NH
RPS = NPAD // NS

_vmesh = plsc.VectorSubcoreMesh(core_axis_name="c", subcore_axis_name="s")


@functools.partial(
    pl.kernel,
    out_type=jax.ShapeDtypeStruct((NC * NPAD,), jnp.float32),
    mesh=_vmesh,
    scratch_types=[
        pltpu.MemorySpace.VMEM_SHARED((NPAD,), jnp.float32),
        pltpu.MemorySpace.VMEM((CHUNKS_H, CW), jnp.int32),
        pltpu.MemorySpace.VMEM((CW,), jnp.float32),
    ],
)
def _deg_kernel(dst_hbm, zeros_hbm, ones_hbm, deg_hbm, acc, idx_v, ones_v):
    c = lax.axis_index("c")
    s = lax.axis_index("s")
    wid = c * NS + s
    pltpu.sync_copy(zeros_hbm, acc.at[pl.ds(s * RPS, RPS)])
    pltpu.sync_copy(ones_hbm, ones_v)
    plsc.subcore_barrier()

    for h in range(NH):
        pltpu.sync_copy(dst_hbm.at[wid, h], idx_v)

        @pl.loop(0, CHUNKS_H)
        def _edge_chunk(j):
            pltpu.sync_copy(ones_v, acc.at[idx_v.at[j]], add=True)

    plsc.subcore_barrier()
    pltpu.sync_copy(acc.at[pl.ds(s * RPS, RPS)],
                    deg_hbm.at[pl.ds(c * NPAD + s * RPS, RPS)])


NB = 2


@functools.partial(
    pl.kernel,
    out_type=jax.ShapeDtypeStruct((NC, NPAD, D), jnp.float32),
    mesh=_vmesh,
    scratch_types=[
        pltpu.MemorySpace.VMEM_SHARED((NPAD, D), jnp.float32),
        pltpu.MemorySpace.VMEM((CHUNKS_H, CW), jnp.int32),
        pltpu.MemorySpace.VMEM((CHUNKS_H, CW), jnp.int32),
        pltpu.MemorySpace.VMEM((NB, CW, D), jnp.float32),
        pltpu.SemaphoreType.DMA((NB,)),
        pltpu.SemaphoreType.DMA((NB,)),
    ],
)
def _agg_kernel(y_hbm, src_hbm, dst_hbm, zrows_hbm, out_hbm,
                acc, src_v, dst_v, rows_v, gsems, ssems):
    c = lax.axis_index("c")
    s = lax.axis_index("s")
    wid = c * NS + s
    pltpu.sync_copy(zrows_hbm, acc.at[pl.ds(s * RPS, RPS), :])
    plsc.subcore_barrier()

    for h in range(NH):
        pltpu.sync_copy(src_hbm.at[wid, h], src_v)
        pltpu.sync_copy(dst_hbm.at[wid, h], dst_v)
        for b in range(NB):
            pltpu.async_copy(y_hbm.at[src_v.at[b]], rows_v.at[b], gsems.at[b])

        @pl.loop(0, CHUNKS_H, step=NB)
        def _edge_chunk(j):
            descs = []
            for b in range(NB):
                pltpu.make_async_copy(
                    y_hbm.at[src_v.at[b]], rows_v.at[b], gsems.at[b]).wait()
                descs.append(pltpu.async_copy(
                    rows_v.at[b], acc.at[dst_v.at[j + b]], ssems.at[b],
                    add=True))
            for b in range(NB):
                descs[b].wait()

                @pl.when(j + NB + b < CHUNKS_H)
                def _refill():
                    pltpu.async_copy(
                        y_hbm.at[src_v.at[j + NB + b]], rows_v.at[b],
                        gsems.at[b])

    plsc.subcore_barrier()
    pltpu.sync_copy(acc.at[pl.ds(s * RPS, RPS), :],
                    out_hbm.at[c, pl.ds(s * RPS, RPS), :])


def _xw_body(x_ref, w_ref, deg_ref, y_ref):
    xw = jnp.dot(x_ref[...], w_ref[...], preferred_element_type=jnp.float32)
    d = deg_ref[0, :] + deg_ref[1, :]
    inv = lax.rsqrt(jnp.maximum(d, 1.0))
    y_ref[...] = xw * inv[:, None]


def _fin_body(acc_ref, deg_ref, out_ref):
    a = acc_ref[0, :, :] + acc_ref[1, :, :]
    d = deg_ref[0, :] + deg_ref[1, :]
    inv = lax.rsqrt(jnp.maximum(d, 1.0))
    out_ref[...] = a * inv[:, None]


_BT = 1024
_BF = 1024

def kernel(X, edge_index, W):
    src = edge_index[0].reshape(NW, NH, CHUNKS_H, CW)
    dst = edge_index[1].reshape(NW, NH, CHUNKS_H, CW)
    zeros1 = jnp.zeros((RPS,), jnp.float32)
    ones1 = jnp.ones((CW,), jnp.float32)
    zrows = jnp.zeros((RPS, D), jnp.float32)

    degp = _deg_kernel(dst, zeros1, ones1).reshape(NC, NPAD)

    y = pl.pallas_call(
        _xw_body,
        grid=(pl.cdiv(N_NODES, _BT),),
        in_specs=[
            pl.BlockSpec((_BT, D), lambda i: (i, 0)),
            pl.BlockSpec((D, D), lambda i: (0, 0)),
            pl.BlockSpec((NC, _BT), lambda i: (0, i)),
        ],
        out_specs=pl.BlockSpec((_BT, D), lambda i: (i, 0)),
        out_shape=jax.ShapeDtypeStruct((N_NODES, D), jnp.float32),
    )(X, W, degp)

    accp = _agg_kernel(y, src, dst, zrows)

    out = pl.pallas_call(
        _fin_body,
        grid=(pl.cdiv(N_NODES, _BF),),
        in_specs=[
            pl.BlockSpec((NC, _BF, D), lambda i: (0, i, 0)),
            pl.BlockSpec((NC, _BF), lambda i: (0, i)),
        ],
        out_specs=pl.BlockSpec((_BF, D), lambda i: (i, 0)),
        out_shape=jax.ShapeDtypeStruct((N_NODES, D), jnp.float32),
    )(accp, degp)

    return out


# speedup vs baseline: 1.3502x; 1.1986x over previous
import functools

import jax
import jax.numpy as jnp
from jax import lax
from jax.experimental import pallas as pl
from jax.experimental.pallas import tpu as pltpu
from jax.experimental.pallas import tpu_sc as plsc

N_NODES = 10000
N_EDGES = 320000
D = 128
NC, NS = 2, 16
NW = NC * NS
NPAD = NS * 640
CW = 128
EPW = N_EDGES // NW
NH = 2
EPH = EPW // NH
NF = EPH // CW
MINI = EPH - NF * CW
NHD = 5
EPHD = EPW // NHD
NFD = EPHD // CW
MINID = EPHD - NFD * CW
RPS = NPAD // NS

_vmesh = plsc.VectorSubcoreMesh(core_axis_name="c", subcore_axis_name="s")


@functools.partial(
    pl.kernel,
    out_type=jax.ShapeDtypeStruct((NC * NPAD,), jnp.float32),
    mesh=_vmesh,
    scratch_types=[
        pltpu.MemorySpace.VMEM_SHARED((NPAD,), jnp.float32),
        pltpu.MemorySpace.VMEM((EPHD,), jnp.int32),
        pltpu.MemorySpace.VMEM((CW,), jnp.float32),
    ],
)
def _deg_kernel(dst_hbm, zeros_hbm, ones_hbm, deg_hbm, acc, idx_v, ones_v):
    c = lax.axis_index("c")
    s = lax.axis_index("s")
    wid = c * NS + s
    pltpu.sync_copy(zeros_hbm, acc.at[pl.ds(s * RPS, RPS)])
    pltpu.sync_copy(ones_hbm, ones_v)
    plsc.subcore_barrier()

    for h in range(NHD):
        pltpu.sync_copy(dst_hbm.at[pl.ds(wid * EPW + h * EPHD, EPHD)], idx_v)

        @pl.loop(0, NFD)
        def _edge_chunk(j):
            pltpu.sync_copy(ones_v, acc.at[idx_v.at[pl.ds(j * CW, CW)]],
                            add=True)

        pltpu.sync_copy(ones_v.at[pl.ds(0, MINID)],
                        acc.at[idx_v.at[pl.ds(NFD * CW, MINID)]], add=True)

    plsc.subcore_barrier()
    pltpu.sync_copy(acc.at[pl.ds(s * RPS, RPS)],
                    deg_hbm.at[pl.ds(c * NPAD + s * RPS, RPS)])


NB = 2


@functools.partial(
    pl.kernel,
    out_type=jax.ShapeDtypeStruct((NC, NPAD, D), jnp.float32),
    mesh=_vmesh,
    scratch_types=[
        pltpu.MemorySpace.VMEM_SHARED((NPAD, D), jnp.float32),
        pltpu.MemorySpace.VMEM((EPH,), jnp.int32),
        pltpu.MemorySpace.VMEM((EPH,), jnp.int32),
        pltpu.MemorySpace.VMEM((NB, CW, D), jnp.float32),
        pltpu.SemaphoreType.DMA((NB,)),
    ],
)
def _agg_kernel(y_hbm, src_hbm, dst_hbm, zrows_hbm, out_hbm,
                acc, src_v, dst_v, rows_v, gsems):
    c = lax.axis_index("c")
    s = lax.axis_index("s")
    wid = c * NS + s
    pltpu.sync_copy(zrows_hbm, acc.at[pl.ds(s * RPS, RPS), :])
    plsc.subcore_barrier()

    for h in range(NH):
        pltpu.sync_copy(src_hbm.at[pl.ds(wid * EPW + h * EPH, EPH)], src_v)
        pltpu.sync_copy(dst_hbm.at[pl.ds(wid * EPW + h * EPH, EPH)], dst_v)
        for b in range(NB):
            pltpu.async_copy(y_hbm.at[src_v.at[pl.ds(b * CW, CW)]],
                             rows_v.at[b], gsems.at[b])

        @pl.loop(0, NF - 1, step=NB)
        def _edge_chunk(j):
            for b in range(NB):
                pltpu.make_async_copy(
                    y_hbm.at[src_v.at[pl.ds(b * CW, CW)]], rows_v.at[b],
                    gsems.at[b]).wait()
                pltpu.sync_copy(rows_v.at[b],
                                acc.at[dst_v.at[pl.ds((j + b) * CW, CW)]],
                                add=True)

                @pl.when(j + NB + b < NF)
                def _refill():
                    pltpu.async_copy(
                        y_hbm.at[src_v.at[pl.ds((j + NB + b) * CW, CW)]],
                        rows_v.at[b], gsems.at[b])

        lb = (NF - 1) % NB
        pltpu.make_async_copy(
            y_hbm.at[src_v.at[pl.ds(lb * CW, CW)]], rows_v.at[lb],
            gsems.at[lb]).wait()
        pltpu.sync_copy(rows_v.at[lb],
                        acc.at[dst_v.at[pl.ds((NF - 1) * CW, CW)]], add=True)
        pltpu.sync_copy(y_hbm.at[src_v.at[pl.ds(NF * CW, MINI)]],
                        rows_v.at[lb, pl.ds(0, MINI), :])
        pltpu.sync_copy(rows_v.at[lb, pl.ds(0, MINI), :],
                        acc.at[dst_v.at[pl.ds(NF * CW, MINI)]], add=True)

    plsc.subcore_barrier()
    pltpu.sync_copy(acc.at[pl.ds(s * RPS, RPS), :],
                    out_hbm.at[c, pl.ds(s * RPS, RPS), :])


def _xw_body(x_ref, w_ref, deg_ref, y_ref):
    xw = jnp.dot(x_ref[...], w_ref[...], preferred_element_type=jnp.float32)
    d = deg_ref[0, :] + deg_ref[1, :]
    inv = lax.rsqrt(jnp.maximum(d, 1.0))
    y_ref[...] = xw * inv[:, None]


def _fin_body(acc_ref, deg_ref, out_ref):
    a = acc_ref[0, :, :] + acc_ref[1, :, :]
    d = deg_ref[0, :] + deg_ref[1, :]
    inv = lax.rsqrt(jnp.maximum(d, 1.0))
    out_ref[...] = a * inv[:, None]


_BT = 1024
_BF = 1024

def kernel(X, edge_index, W):
    src = edge_index[0]
    dst = edge_index[1]
    zeros1 = jnp.zeros((RPS,), jnp.float32)
    ones1 = jnp.ones((CW,), jnp.float32)
    zrows = jnp.zeros((RPS, D), jnp.float32)

    degp = _deg_kernel(dst, zeros1, ones1).reshape(NC, NPAD)

    y = pl.pallas_call(
        _xw_body,
        grid=(pl.cdiv(N_NODES, _BT),),
        in_specs=[
            pl.BlockSpec((_BT, D), lambda i: (i, 0)),
            pl.BlockSpec((D, D), lambda i: (0, 0)),
            pl.BlockSpec((NC, _BT), lambda i: (0, i)),
        ],
        out_specs=pl.BlockSpec((_BT, D), lambda i: (i, 0)),
        out_shape=jax.ShapeDtypeStruct((N_NODES, D), jnp.float32),
    )(X, W, degp)

    accp = _agg_kernel(y, src, dst, zrows)

    out = pl.pallas_call(
        _fin_body,
        grid=(pl.cdiv(N_NODES, _BF),),
        in_specs=[
            pl.BlockSpec((NC, _BF, D), lambda i: (0, i, 0)),
            pl.BlockSpec((NC, _BF), lambda i: (0, i)),
        ],
        out_specs=pl.BlockSpec((_BF, D), lambda i: (i, 0)),
        out_shape=jax.ShapeDtypeStruct((N_NODES, D), jnp.float32),
    )(accp, degp)

    return out
